# Initial kernel scaffold; baseline (speedup 1.0000x reference)
import jax
import jax.numpy as jnp
from jax.experimental import pallas as pl


def kernel(x, edge_index, W1, b1, W2, b2):
    raise NotImplementedError("write your pallas kernel here")


# speedup vs baseline: 24.1756x; 24.1756x over previous
import functools

import jax
import jax.numpy as jnp
from jax import lax
from jax.experimental import pallas as pl
from jax.experimental.pallas import tpu as pltpu
from jax.experimental.pallas import tpu_sc as plsc

N = 10000
E = 160000
D_HID = 16

NC = 2
NS = 16
NW = NC * NS
EPW = E // NW
CW = 125
NCHUNK = EPW // CW
NPAD = 10240
RPT = NPAD // NS

_sc_mesh = plsc.VectorSubcoreMesh(core_axis_name="c", subcore_axis_name="s")


def _deg_body(dstr, zeros, ones, out, dst_v, ones_v, acc, sem):
    c = lax.axis_index("c")
    s = lax.axis_index("s")
    wid = c * NS + s
    pltpu.sync_copy(zeros.at[pl.ds(s * RPT, RPT)], acc.at[pl.ds(s * RPT, RPT)])
    pltpu.sync_copy(dstr.at[wid], dst_v)
    pltpu.sync_copy(ones, ones_v)
    plsc.subcore_barrier()

    def body(j, carry):
        pltpu.async_copy(ones_v, acc.at[dst_v.at[j]], sem, add=True).wait()
        return carry

    lax.fori_loop(0, NCHUNK, body, 0)
    plsc.subcore_barrier()
    pltpu.sync_copy(acc.at[pl.ds(s * RPT, RPT)], out.at[c, pl.ds(s * RPT, RPT)])


_deg_call = functools.partial(
    pl.kernel,
    _deg_body,
    out_type=jax.ShapeDtypeStruct((NC, NPAD, D_HID), jnp.float32),
    mesh=_sc_mesh,
    compiler_params=pltpu.CompilerParams(use_tc_tiling_on_sc=False),
    scratch_types=[
        pltpu.VMEM((NCHUNK, CW), jnp.int32),
        pltpu.VMEM((CW, D_HID), jnp.float32),
        pltpu.VMEM_SHARED((NPAD, D_HID), jnp.float32),
        pltpu.SemaphoreType.DMA,
    ],
)()


def _scat_body(srcr, dstr, y, zeros, out, src_v, dst_v, rows_v, acc, sem, sem2):
    c = lax.axis_index("c")
    s = lax.axis_index("s")
    wid = c * NS + s
    pltpu.sync_copy(zeros.at[pl.ds(s * RPT, RPT)], acc.at[pl.ds(s * RPT, RPT)])
    pltpu.sync_copy(srcr.at[wid], src_v)
    pltpu.sync_copy(dstr.at[wid], dst_v)
    plsc.subcore_barrier()

    def body(j, carry):
        pltpu.async_copy(y.at[src_v.at[j]], rows_v.at[j], sem).wait()
        pltpu.async_copy(rows_v.at[j], acc.at[dst_v.at[j]], sem2, add=True).wait()
        return carry

    lax.fori_loop(0, NCHUNK, body, 0)
    plsc.subcore_barrier()
    pltpu.sync_copy(acc.at[pl.ds(s * RPT, RPT)], out.at[c, pl.ds(s * RPT, RPT)])


_scat_call = functools.partial(
    pl.kernel,
    _scat_body,
    out_type=jax.ShapeDtypeStruct((NC, NPAD, D_HID), jnp.float32),
    mesh=_sc_mesh,
    compiler_params=pltpu.CompilerParams(use_tc_tiling_on_sc=False),
    scratch_types=[
        pltpu.VMEM((NCHUNK, CW), jnp.int32),
        pltpu.VMEM((NCHUNK, CW), jnp.int32),
        pltpu.VMEM((NCHUNK, CW, D_HID), jnp.float32),
        pltpu.VMEM_SHARED((NPAD, D_HID), jnp.float32),
        pltpu.SemaphoreType.DMA,
        pltpu.SemaphoreType.DMA,
    ],
)()


def _tc1_body(degp_ref, x_ref, w1_ref, y1_ref, dinv_ref):
    deg = degp_ref[0, :N, 0:1] + degp_ref[1, :N, 0:1] + 1.0
    dinv = lax.rsqrt(deg)
    xw = jnp.dot(x_ref[...], w1_ref[...], preferred_element_type=jnp.float32)
    y1_ref[...] = xw * dinv
    dinv_ref[...] = dinv


def _tc2_body(s1p_ref, y1_ref, dinv_ref, b1_ref, y2_ref):
    dinv = dinv_ref[...]
    agg = (s1p_ref[0, :N] + s1p_ref[1, :N] + y1_ref[...]) * dinv + b1_ref[...]
    h = jnp.maximum(agg, 0.0)
    y2_ref[...] = h * dinv


def _tc3_body(s2p_ref, y2_ref, dinv_ref, w2_ref, b2_ref, out_ref):
    t = (s2p_ref[0, :N] + s2p_ref[1, :N] + y2_ref[...]) * dinv_ref[...]
    z = jnp.dot(t, w2_ref[...], preferred_element_type=jnp.float32) + b2_ref[...]
    out_ref[...] = jax.nn.sigmoid(z)


def kernel(x, edge_index, W1, b1, W2, b2):
    src_r = edge_index[0].reshape(NW, NCHUNK, CW)
    dst_r = edge_index[1].reshape(NW, NCHUNK, CW)
    zeros16 = jnp.zeros((NPAD, D_HID), jnp.float32)
    ones_row = jnp.ones((CW, D_HID), jnp.float32)

    degp = _deg_call(dst_r, zeros16, ones_row)

    y1, dinv = pl.pallas_call(
        _tc1_body,
        out_shape=[
            jax.ShapeDtypeStruct((N, D_HID), jnp.float32),
            jax.ShapeDtypeStruct((N, 1), jnp.float32),
        ],
    )(degp, x, W1)

    s1p = _scat_call(src_r, dst_r, y1, zeros16)

    y2 = pl.pallas_call(
        _tc2_body,
        out_shape=jax.ShapeDtypeStruct((N, D_HID), jnp.float32),
    )(s1p, y1, dinv, b1.reshape(1, D_HID))

    s2p = _scat_call(src_r, dst_r, y2, zeros16)

    out = pl.pallas_call(
        _tc3_body,
        out_shape=jax.ShapeDtypeStruct((N, W2.shape[1]), jnp.float32),
    )(s2p, y2, dinv, W2, b2.reshape(1, W2.shape[1]))
    return out


# speedup vs baseline: 33.2711x; 1.3762x over previous
import functools

import jax
import jax.numpy as jnp
from jax import lax
from jax.experimental import pallas as pl
from jax.experimental.pallas import tpu as pltpu
from jax.experimental.pallas import tpu_sc as plsc

N = 10000
E = 160000
D_HID = 16

NC = 2
NS = 16
NW = NC * NS
EPW = E // NW
CW = 125
NCHUNK = EPW // CW
NPAD = 10240
RPT = NPAD // NS

_sc_mesh = plsc.VectorSubcoreMesh(core_axis_name="c", subcore_axis_name="s")


def _deg_body(dstr, zeros, ones, out, dst_v, ones_v, acc, sem):
    c = lax.axis_index("c")
    s = lax.axis_index("s")
    wid = c * NS + s
    pltpu.sync_copy(zeros.at[pl.ds(s * RPT, RPT)], acc.at[pl.ds(s * RPT, RPT)])
    pltpu.sync_copy(dstr.at[wid], dst_v)
    pltpu.sync_copy(ones, ones_v)
    plsc.subcore_barrier()

    def fire(j, carry):
        pltpu.async_copy(ones_v, acc.at[dst_v.at[j]], sem, add=True)
        return carry

    lax.fori_loop(0, NCHUNK, fire, 0)

    def drain(j, carry):
        pltpu.make_async_copy(ones_v, acc.at[dst_v.at[j]], sem).wait()
        return carry

    lax.fori_loop(0, NCHUNK, drain, 0)
    plsc.subcore_barrier()
    pltpu.sync_copy(acc.at[pl.ds(s * RPT, RPT)], out.at[c, pl.ds(s * RPT, RPT)])


_deg_call = functools.partial(
    pl.kernel,
    _deg_body,
    out_type=jax.ShapeDtypeStruct((NC, NPAD, D_HID), jnp.float32),
    mesh=_sc_mesh,
    compiler_params=pltpu.CompilerParams(use_tc_tiling_on_sc=False),
    scratch_types=[
        pltpu.VMEM((NCHUNK, CW), jnp.int32),
        pltpu.VMEM((CW, D_HID), jnp.float32),
        pltpu.VMEM_SHARED((NPAD, D_HID), jnp.float32),
        pltpu.SemaphoreType.DMA,
    ],
)()


def _scat_body(srcr, dstr, y, zeros, out, src_v, dst_v, rows_v, acc, sem, sem2):
    c = lax.axis_index("c")
    s = lax.axis_index("s")
    wid = c * NS + s
    pltpu.sync_copy(zeros.at[pl.ds(s * RPT, RPT)], acc.at[pl.ds(s * RPT, RPT)])
    pltpu.sync_copy(srcr.at[wid], src_v)
    pltpu.sync_copy(dstr.at[wid], dst_v)
    plsc.subcore_barrier()

    def fire_g(j, carry):
        pltpu.async_copy(y.at[src_v.at[j]], rows_v.at[j], sem)
        return carry

    lax.fori_loop(0, NCHUNK, fire_g, 0)

    def drain_g(j, carry):
        pltpu.make_async_copy(y.at[src_v.at[j]], rows_v.at[j], sem).wait()
        return carry

    lax.fori_loop(0, NCHUNK, drain_g, 0)

    def fire_s(j, carry):
        pltpu.async_copy(rows_v.at[j], acc.at[dst_v.at[j]], sem2, add=True)
        return carry

    lax.fori_loop(0, NCHUNK, fire_s, 0)

    def drain_s(j, carry):
        pltpu.make_async_copy(rows_v.at[j], acc.at[dst_v.at[j]], sem2).wait()
        return carry

    lax.fori_loop(0, NCHUNK, drain_s, 0)
    plsc.subcore_barrier()
    pltpu.sync_copy(acc.at[pl.ds(s * RPT, RPT)], out.at[c, pl.ds(s * RPT, RPT)])


_scat_call = functools.partial(
    pl.kernel,
    _scat_body,
    out_type=jax.ShapeDtypeStruct((NC, NPAD, D_HID), jnp.float32),
    mesh=_sc_mesh,
    compiler_params=pltpu.CompilerParams(use_tc_tiling_on_sc=False),
    scratch_types=[
        pltpu.VMEM((NCHUNK, CW), jnp.int32),
        pltpu.VMEM((NCHUNK, CW), jnp.int32),
        pltpu.VMEM((NCHUNK, CW, D_HID), jnp.float32),
        pltpu.VMEM_SHARED((NPAD, D_HID), jnp.float32),
        pltpu.SemaphoreType.DMA,
        pltpu.SemaphoreType.DMA,
    ],
)()


def _tc1_body(degp_ref, x_ref, w1_ref, y1_ref, dinv_ref):
    deg = degp_ref[0, :N, 0:1] + degp_ref[1, :N, 0:1] + 1.0
    dinv = lax.rsqrt(deg)
    xw = jnp.dot(x_ref[...], w1_ref[...], preferred_element_type=jnp.float32)
    y1_ref[...] = xw * dinv
    dinv_ref[...] = dinv


def _tc2_body(s1p_ref, y1_ref, dinv_ref, b1_ref, y2_ref):
    dinv = dinv_ref[...]
    agg = (s1p_ref[0, :N] + s1p_ref[1, :N] + y1_ref[...]) * dinv + b1_ref[...]
    h = jnp.maximum(agg, 0.0)
    y2_ref[...] = h * dinv


def _tc3_body(s2p_ref, y2_ref, dinv_ref, w2_ref, b2_ref, out_ref):
    t = (s2p_ref[0, :N] + s2p_ref[1, :N] + y2_ref[...]) * dinv_ref[...]
    z = jnp.dot(t, w2_ref[...], preferred_element_type=jnp.float32) + b2_ref[...]
    out_ref[...] = jax.nn.sigmoid(z)


def kernel(x, edge_index, W1, b1, W2, b2):
    src_r = edge_index[0].reshape(NW, NCHUNK, CW)
    dst_r = edge_index[1].reshape(NW, NCHUNK, CW)
    zeros16 = jnp.zeros((NPAD, D_HID), jnp.float32)
    ones_row = jnp.ones((CW, D_HID), jnp.float32)

    degp = _deg_call(dst_r, zeros16, ones_row)

    y1, dinv = pl.pallas_call(
        _tc1_body,
        out_shape=[
            jax.ShapeDtypeStruct((N, D_HID), jnp.float32),
            jax.ShapeDtypeStruct((N, 1), jnp.float32),
        ],
    )(degp, x, W1)

    s1p = _scat_call(src_r, dst_r, y1, zeros16)

    y2 = pl.pallas_call(
        _tc2_body,
        out_shape=jax.ShapeDtypeStruct((N, D_HID), jnp.float32),
    )(s1p, y1, dinv, b1.reshape(1, D_HID))

    s2p = _scat_call(src_r, dst_r, y2, zeros16)

    out = pl.pallas_call(
        _tc3_body,
        out_shape=jax.ShapeDtypeStruct((N, W2.shape[1]), jnp.float32),
    )(s2p, y2, dinv, W2, b2.reshape(1, W2.shape[1]))
    return out


# speedup vs baseline: 53.8424x; 1.6183x over previous
import functools

import jax
import jax.numpy as jnp
from jax import lax
from jax.experimental import pallas as pl
from jax.experimental.pallas import tpu as pltpu
from jax.experimental.pallas import tpu_sc as plsc

N = 10000
E = 160000
D_HID = 16

NC = 2
NS = 16
NW = NC * NS
EPW = E // NW
CW = 125
NCHUNK = EPW // CW
BUF = 40
ROUNDS = NCHUNK // BUF
NPAD = 10240
RPT = NPAD // NS

_sc_mesh = plsc.VectorSubcoreMesh(core_axis_name="c", subcore_axis_name="s")


def _deg_body(dstr, zeros, ones, out, dst_v, ones_v, acc, sem):
    c = lax.axis_index("c")
    s = lax.axis_index("s")
    wid = c * NS + s
    pltpu.sync_copy(zeros.at[pl.ds(s * RPT, RPT)], acc.at[pl.ds(s * RPT, RPT)])
    pltpu.sync_copy(dstr.at[wid], dst_v)
    pltpu.sync_copy(ones, ones_v)
    plsc.subcore_barrier()

    def fire(j, carry):
        pltpu.async_copy(ones_v, acc.at[dst_v.at[j]], sem, add=True)
        return carry

    lax.fori_loop(0, NCHUNK, fire, 0)

    def drain(j, carry):
        pltpu.make_async_copy(ones_v, acc.at[dst_v.at[j]], sem).wait()
        return carry

    lax.fori_loop(0, NCHUNK, drain, 0)
    plsc.subcore_barrier()
    pltpu.sync_copy(acc.at[pl.ds(s * RPT, RPT)], out.at[c, pl.ds(s * RPT, RPT)])


_deg_call = functools.partial(
    pl.kernel,
    _deg_body,
    out_type=jax.ShapeDtypeStruct((NC, NPAD, D_HID), jnp.float32),
    mesh=_sc_mesh,
    compiler_params=pltpu.CompilerParams(use_tc_tiling_on_sc=False),
    scratch_types=[
        pltpu.VMEM((NCHUNK, CW), jnp.int32),
        pltpu.VMEM((CW, D_HID), jnp.float32),
        pltpu.VMEM_SHARED((NPAD, D_HID), jnp.float32),
        pltpu.SemaphoreType.DMA,
    ],
)()


def _scat_body(srcr, dstr, y, zeros, out, src_v, dst_v, rows_v, acc, sem, sem2):
    c = lax.axis_index("c")
    s = lax.axis_index("s")
    wid = c * NS + s
    pltpu.sync_copy(zeros.at[pl.ds(s * RPT, RPT)], acc.at[pl.ds(s * RPT, RPT)])
    pltpu.sync_copy(srcr.at[wid], src_v)
    pltpu.sync_copy(dstr.at[wid], dst_v)
    plsc.subcore_barrier()

    for r in range(ROUNDS):
        base = r * BUF

        def fire_g(j, carry):
            pltpu.async_copy(y.at[src_v.at[base + j]], rows_v.at[j], sem)
            return carry

        lax.fori_loop(0, BUF, fire_g, 0)

        def drain_fire(j, carry):
            pltpu.make_async_copy(y.at[src_v.at[base + j]], rows_v.at[j], sem).wait()
            pltpu.async_copy(rows_v.at[j], acc.at[dst_v.at[base + j]], sem2, add=True)
            return carry

        lax.fori_loop(0, BUF, drain_fire, 0)

        def drain_s(j, carry):
            pltpu.make_async_copy(rows_v.at[j], acc.at[dst_v.at[base + j]], sem2).wait()
            return carry

        lax.fori_loop(0, BUF, drain_s, 0)
    plsc.subcore_barrier()
    pltpu.sync_copy(acc.at[pl.ds(s * RPT, RPT)], out.at[c, pl.ds(s * RPT, RPT)])


_scat_call = functools.partial(
    pl.kernel,
    _scat_body,
    out_type=jax.ShapeDtypeStruct((NC, NPAD, D_HID), jnp.float32),
    mesh=_sc_mesh,
    compiler_params=pltpu.CompilerParams(use_tc_tiling_on_sc=False),
    scratch_types=[
        pltpu.VMEM((NCHUNK, CW), jnp.int32),
        pltpu.VMEM((NCHUNK, CW), jnp.int32),
        pltpu.VMEM((BUF, CW, D_HID), jnp.float32),
        pltpu.VMEM_SHARED((NPAD, D_HID), jnp.float32),
        pltpu.SemaphoreType.DMA,
        pltpu.SemaphoreType.DMA,
    ],
)()


NPK = NPAD // 8
NB = N // 8


def _tc0_body(x_ref, w1_ref, u1_ref):
    for k in range(8):
        u1_ref[:, 16 * k:16 * (k + 1)] = jnp.dot(
            x_ref[NB * k:NB * (k + 1), :], w1_ref[...],
            preferred_element_type=jnp.float32)


def _tc1_body(degp_ref, u1_ref, y1_ref, dinv_ref):
    deg = degp_ref[0] + degp_ref[1] + 1.0
    dinv = lax.rsqrt(deg)
    dinv_ref[...] = dinv
    y1_ref[...] = u1_ref[...] * dinv[:NB]


def _tc2_body(s1p_ref, y1_ref, dinv_ref, b1_ref, y2_ref):
    dinv = dinv_ref[:NB]
    agg = (s1p_ref[0, :NB] + s1p_ref[1, :NB] + y1_ref[...]) * dinv + b1_ref[...]
    h = jnp.maximum(agg, 0.0)
    y2_ref[...] = h * dinv


def _tc3_body(s2p_ref, y2_ref, dinv_ref, w2_ref, b2_ref, out_ref):
    tp = (s2p_ref[0, :NB] + s2p_ref[1, :NB] + y2_ref[...]) * dinv_ref[:NB]
    for k in range(8):
        z = jnp.dot(tp[:, 16 * k:16 * (k + 1)], w2_ref[...],
                    preferred_element_type=jnp.float32) + b2_ref[...]
        out_ref[NB * k:NB * (k + 1), :] = jax.nn.sigmoid(z)


def kernel(x, edge_index, W1, b1, W2, b2):
    ep = (edge_index % NB) * 8 + edge_index // NB
    src_r = ep[0].reshape(NW, NCHUNK, CW)
    dst_r = ep[1].reshape(NW, NCHUNK, CW)
    zeros16 = jnp.zeros((NPAD, D_HID), jnp.float32)
    ones_row = jnp.ones((CW, D_HID), jnp.float32)
    b1p = jnp.tile(b1, 8).reshape(1, 128)

    u1p = pl.pallas_call(
        _tc0_body,
        out_shape=jax.ShapeDtypeStruct((NB, 128), jnp.float32),
    )(x, W1)

    degp = _deg_call(dst_r, zeros16, ones_row)

    y1p, dinvp = pl.pallas_call(
        _tc1_body,
        out_shape=[
            jax.ShapeDtypeStruct((NB, 128), jnp.float32),
            jax.ShapeDtypeStruct((NPK, 128), jnp.float32),
        ],
    )(degp.reshape(NC, NPK, 128), u1p)

    s1p = _scat_call(src_r, dst_r, y1p.reshape(N, D_HID), zeros16)

    y2p = pl.pallas_call(
        _tc2_body,
        out_shape=jax.ShapeDtypeStruct((NB, 128), jnp.float32),
    )(s1p.reshape(NC, NPK, 128), y1p, dinvp, b1p)

    s2p = _scat_call(src_r, dst_r, y2p.reshape(N, D_HID), zeros16)

    out = pl.pallas_call(
        _tc3_body,
        out_shape=jax.ShapeDtypeStruct((N, W2.shape[1]), jnp.float32),
    )(s2p.reshape(NC, NPK, 128), y2p, dinvp, W2, b2.reshape(1, W2.shape[1]))
    return out
